# baseline (speedup 1.0000x reference)
import functools
import jax
import jax.numpy as jnp
from jax import lax
from jax.experimental import pallas as pl
from jax.experimental.pallas import tpu as pltpu
from jax.experimental.pallas import tpu_sc as plsc

N = 1024
H = 64
B = 8
TB = 32
SC_BASE = 896
NW = 32
RPW = (N - SC_BASE) // NW
CH = 128
CPR = N // CH
TOTC = RPW * CPR

_mesh = plsc.VectorSubcoreMesh(core_axis_name="c", subcore_axis_name="s")


def _splat(x):
    return jnp.full((16,), x, jnp.int32)


@functools.partial(
    pl.kernel,
    out_type=jax.ShapeDtypeStruct(((N - SC_BASE) * H,), jnp.float32),
    mesh=_mesh,
    compiler_params=pltpu.CompilerParams(needs_layout_passes=False),
    scratch_types=[
        pltpu.VMEM((N * H,), jnp.float32),
        pltpu.VMEM((RPW * N,), jnp.int32),
        pltpu.VMEM((2, CH, H), jnp.float32),
        pltpu.VMEM((RPW * H,), jnp.float32),
        pltpu.SemaphoreType.DMA,
    ],
)
def _sc_agg(nf_hbm, adj_hbm, e_hbm, out_hbm,
            nf_v, adj_v, stage_v, out_v, sem):
    wid = lax.axis_index("s") * 2 + lax.axis_index("c")
    row0 = SC_BASE + wid * RPW

    dnums = lax.GatherDimensionNumbers(
        offset_dims=(), collapsed_slice_dims=(0,), start_index_map=(0,))

    def take16(v, idx):
        return lax.gather(v, idx[:, None], dnums, slice_sizes=(1,),
                          mode=lax.GatherScatterMode.PROMISE_IN_BOUNDS)

    pltpu.sync_copy(nf_hbm, nf_v)
    pltpu.sync_copy(adj_hbm.at[pl.ds(row0 * N, RPW * N)], adj_v)

    def _issue(gc):
        i_local = gc // CPR
        c = gc % CPR
        pltpu.async_copy(
            e_hbm.at[pl.ds((row0 + i_local) * N + c * CH, CH)],
            stage_v.at[gc % 2], sem)

    _issue(0)

    def chunk_body(gc, acc4):
        i_local = gc // CPR
        c = gc % CPR
        b = gc % 2
        lax.cond(gc + 1 < TOTC, lambda: _issue(gc + 1), lambda: None)
        pltpu.make_async_copy(
            e_hbm.at[pl.ds(0, CH)], stage_v.at[b], sem).wait()
        jbase = c * CH

        def eb_body(eb, acc4):
            wv = adj_v[pl.ds(i_local * N + jbase + eb * 16, 16)]
            wf = jnp.where(wv != 0, 1.0, 0.0).astype(jnp.float32)
            for k in range(16):
                w = take16(wf, _splat(k))
                e = eb * 16 + k
                j = jbase + e
                new = []
                for hb in range(4):
                    ev = stage_v[b, e, pl.ds(hb * 16, 16)]
                    nfv = nf_v[pl.ds(j * H + hb * 16, 16)]
                    new.append(acc4[hb] + w * ev * nfv)
                acc4 = tuple(new)
            return acc4

        acc4 = lax.fori_loop(0, CH // 16, eb_body, acc4)

        def flush(a4, i_local=i_local):
            for hb in range(4):
                out_v[pl.ds(i_local * H + hb * 16, 16)] = a4[hb]
            return (jnp.zeros((16,), jnp.float32),) * 4

        return lax.cond(c == CPR - 1, flush, lambda a4: a4, acc4)

    lax.fori_loop(0, TOTC, chunk_body,
                  (jnp.zeros((16,), jnp.float32),) * 4)

    pltpu.sync_copy(out_v, out_hbm.at[pl.ds((row0 - SC_BASE) * H, RPW * H)])


def _tc_kernel(e_ref, nft_ref, sa_ref, h_ref,
               wih_ref, whh_ref, bih_ref, bhh_ref, out_ref):
    aggs = []
    for r0 in range(0, TB, B):
        q = e_ref[pl.ds(r0 * N, B * N), :] * nft_ref[:]
        aggs.append(jnp.dot(sa_ref[pl.ds(r0, B), :], q,
                            preferred_element_type=jnp.float32))
    agg = jnp.concatenate(aggs, axis=0)
    h = h_ref[:]
    gi = jnp.dot(agg, wih_ref[:], preferred_element_type=jnp.float32) + bih_ref[:]
    gh = jnp.dot(h, whh_ref[:], preferred_element_type=jnp.float32) + bhh_ref[:]
    r = jax.nn.sigmoid(gi[:, :H] + gh[:, :H])
    z = jax.nn.sigmoid(gi[:, H:2 * H] + gh[:, H:2 * H])
    n = jnp.tanh(gi[:, 2 * H:] + r * gh[:, 2 * H:])
    out_ref[:] = (1.0 - z) * n + z * h


def _gru_kernel(agg_ref, h_ref, wih_ref, whh_ref, bih_ref, bhh_ref, out_ref):
    gi = jnp.dot(agg_ref[:], wih_ref[:], preferred_element_type=jnp.float32) + bih_ref[:]
    gh = jnp.dot(h_ref[:], whh_ref[:], preferred_element_type=jnp.float32) + bhh_ref[:]
    r = jax.nn.sigmoid(gi[:, :H] + gh[:, :H])
    z = jax.nn.sigmoid(gi[:, H:2 * H] + gh[:, H:2 * H])
    n = jnp.tanh(gi[:, 2 * H:] + r * gh[:, 2 * H:])
    out_ref[:] = (1.0 - z) * n + z * h_ref[:]


def kernel(node_features, adjacency_matrix, edge_embedding,
           weight_ih, weight_hh, bias_ih, bias_hh):
    wih_t = weight_ih.T
    whh_t = weight_hh.T
    bih = bias_ih.reshape(1, 3 * H)
    bhh = bias_hh.reshape(1, 3 * H)

    agg_sc = _sc_agg(node_features.reshape(N * H),
                     adjacency_matrix.reshape(N * N),
                     edge_embedding).reshape(N - SC_BASE, H)
    out_sc = pl.pallas_call(
        _gru_kernel,
        out_shape=jax.ShapeDtypeStruct((N - SC_BASE, H), jnp.float32),
    )(agg_sc, node_features[SC_BASE:], wih_t, whh_t, bih, bhh)

    a_f = (adjacency_matrix[:SC_BASE] != 0).astype(jnp.float32)
    eye = jnp.eye(B, dtype=jnp.float32)
    sa = (a_f.reshape(SC_BASE // B, B, 1, N) * eye[None, :, :, None]
          ).reshape(SC_BASE, B * N)
    nft = jnp.tile(node_features, (B, 1))

    out_tc = pl.pallas_call(
        _tc_kernel,
        grid=(SC_BASE // TB,),
        in_specs=[
            pl.BlockSpec((TB * N, H), lambda i: (i, 0)),
            pl.BlockSpec((B * N, H), lambda i: (0, 0)),
            pl.BlockSpec((TB, B * N), lambda i: (i, 0)),
            pl.BlockSpec((TB, H), lambda i: (i, 0)),
            pl.BlockSpec((H, 3 * H), lambda i: (0, 0)),
            pl.BlockSpec((H, 3 * H), lambda i: (0, 0)),
            pl.BlockSpec((1, 3 * H), lambda i: (0, 0)),
            pl.BlockSpec((1, 3 * H), lambda i: (0, 0)),
        ],
        out_specs=pl.BlockSpec((TB, H), lambda i: (i, 0)),
        out_shape=jax.ShapeDtypeStruct((SC_BASE, H), jnp.float32),
    )(edge_embedding, nft, sa, node_features[:SC_BASE],
      wih_t, whh_t, bih, bhh)

    return jnp.concatenate([out_tc, out_sc], axis=0)


# speedup vs baseline: 1.0266x; 1.0266x over previous
import functools
import jax
import jax.numpy as jnp
from jax import lax
from jax.experimental import pallas as pl
from jax.experimental.pallas import tpu as pltpu
from jax.experimental.pallas import tpu_sc as plsc

N = 1024
H = 64
B = 8
TB = 32
SC_BASE = 832
NW = 32
RPW = (N - SC_BASE) // NW
CH = 128
CPR = N // CH
TOTC = RPW * CPR

_mesh = plsc.VectorSubcoreMesh(core_axis_name="c", subcore_axis_name="s")


def _splat(x):
    return jnp.full((16,), x, jnp.int32)


@functools.partial(
    pl.kernel,
    out_type=jax.ShapeDtypeStruct(((N - SC_BASE) * H,), jnp.float32),
    mesh=_mesh,
    compiler_params=pltpu.CompilerParams(needs_layout_passes=False),
    scratch_types=[
        pltpu.VMEM((N * H,), jnp.float32),
        pltpu.VMEM((RPW * N,), jnp.int32),
        pltpu.VMEM((2, CH, H), jnp.float32),
        pltpu.VMEM((RPW * H,), jnp.float32),
        pltpu.SemaphoreType.DMA,
    ],
)
def _sc_agg(nf_hbm, adj_hbm, e_hbm, out_hbm,
            nf_v, adj_v, stage_v, out_v, sem):
    wid = lax.axis_index("s") * 2 + lax.axis_index("c")
    row0 = SC_BASE + wid * RPW

    dnums = lax.GatherDimensionNumbers(
        offset_dims=(), collapsed_slice_dims=(0,), start_index_map=(0,))

    def take16(v, idx):
        return lax.gather(v, idx[:, None], dnums, slice_sizes=(1,),
                          mode=lax.GatherScatterMode.PROMISE_IN_BOUNDS)

    pltpu.sync_copy(nf_hbm, nf_v)
    pltpu.sync_copy(adj_hbm.at[pl.ds(row0 * N, RPW * N)], adj_v)

    def _issue(gc):
        i_local = gc // CPR
        c = gc % CPR
        pltpu.async_copy(
            e_hbm.at[pl.ds((row0 + i_local) * N + c * CH, CH)],
            stage_v.at[gc % 2], sem)

    _issue(0)

    def chunk_body(gc, acc4):
        i_local = gc // CPR
        c = gc % CPR
        b = gc % 2
        lax.cond(gc + 1 < TOTC, lambda: _issue(gc + 1), lambda: None)
        pltpu.make_async_copy(
            e_hbm.at[pl.ds(0, CH)], stage_v.at[b], sem).wait()
        jbase = c * CH

        def eb_body(eb, acc4):
            wv = adj_v[pl.ds(i_local * N + jbase + eb * 16, 16)]
            wf = jnp.where(wv != 0, 1.0, 0.0).astype(jnp.float32)
            for k in range(16):
                w = take16(wf, _splat(k))
                e = eb * 16 + k
                j = jbase + e
                new = []
                for hb in range(4):
                    ev = stage_v[b, e, pl.ds(hb * 16, 16)]
                    nfv = nf_v[pl.ds(j * H + hb * 16, 16)]
                    new.append(acc4[hb] + w * ev * nfv)
                acc4 = tuple(new)
            return acc4

        acc4 = lax.fori_loop(0, CH // 16, eb_body, acc4)

        def flush(a4, i_local=i_local):
            for hb in range(4):
                out_v[pl.ds(i_local * H + hb * 16, 16)] = a4[hb]
            return (jnp.zeros((16,), jnp.float32),) * 4

        return lax.cond(c == CPR - 1, flush, lambda a4: a4, acc4)

    lax.fori_loop(0, TOTC, chunk_body,
                  (jnp.zeros((16,), jnp.float32),) * 4)

    pltpu.sync_copy(out_v, out_hbm.at[pl.ds((row0 - SC_BASE) * H, RPW * H)])


def _tc_kernel(e_ref, nft_ref, a_ref, h_ref,
               wih_ref, whh_ref, bih_ref, bhh_ref, out_ref):
    col = jax.lax.broadcasted_iota(jnp.int32, (B, B * N), 1) // N
    row = jax.lax.broadcasted_iota(jnp.int32, (B, B * N), 0)
    diag = (col == row).astype(jnp.float32)
    aggs = []
    for r0 in range(0, TB, B):
        q = e_ref[pl.ds(r0 * N, B * N), :] * nft_ref[:]
        sa8 = jnp.tile(a_ref[pl.ds(r0, B), :], (1, B)) * diag
        aggs.append(jnp.dot(sa8, q, preferred_element_type=jnp.float32))
    agg = jnp.concatenate(aggs, axis=0)
    h = h_ref[:]
    gi = jnp.dot(agg, wih_ref[:], preferred_element_type=jnp.float32) + bih_ref[:]
    gh = jnp.dot(h, whh_ref[:], preferred_element_type=jnp.float32) + bhh_ref[:]
    r = jax.nn.sigmoid(gi[:, :H] + gh[:, :H])
    z = jax.nn.sigmoid(gi[:, H:2 * H] + gh[:, H:2 * H])
    n = jnp.tanh(gi[:, 2 * H:] + r * gh[:, 2 * H:])
    out_ref[:] = (1.0 - z) * n + z * h


def _gru_kernel(agg_ref, h_ref, wih_ref, whh_ref, bih_ref, bhh_ref, out_ref):
    gi = jnp.dot(agg_ref[:], wih_ref[:], preferred_element_type=jnp.float32) + bih_ref[:]
    gh = jnp.dot(h_ref[:], whh_ref[:], preferred_element_type=jnp.float32) + bhh_ref[:]
    r = jax.nn.sigmoid(gi[:, :H] + gh[:, :H])
    z = jax.nn.sigmoid(gi[:, H:2 * H] + gh[:, H:2 * H])
    n = jnp.tanh(gi[:, 2 * H:] + r * gh[:, 2 * H:])
    out_ref[:] = (1.0 - z) * n + z * h_ref[:]


def kernel(node_features, adjacency_matrix, edge_embedding,
           weight_ih, weight_hh, bias_ih, bias_hh):
    wih_t = weight_ih.T
    whh_t = weight_hh.T
    bih = bias_ih.reshape(1, 3 * H)
    bhh = bias_hh.reshape(1, 3 * H)

    agg_sc = _sc_agg(node_features.reshape(N * H),
                     adjacency_matrix.reshape(N * N),
                     edge_embedding).reshape(N - SC_BASE, H)
    out_sc = pl.pallas_call(
        _gru_kernel,
        out_shape=jax.ShapeDtypeStruct((N - SC_BASE, H), jnp.float32),
    )(agg_sc, node_features[SC_BASE:], wih_t, whh_t, bih, bhh)

    a_f = (adjacency_matrix[:SC_BASE] != 0).astype(jnp.float32)
    nft = jnp.tile(node_features, (B, 1))

    out_tc = pl.pallas_call(
        _tc_kernel,
        grid=(SC_BASE // TB,),
        in_specs=[
            pl.BlockSpec((TB * N, H), lambda i: (i, 0)),
            pl.BlockSpec((B * N, H), lambda i: (0, 0)),
            pl.BlockSpec((TB, N), lambda i: (i, 0)),
            pl.BlockSpec((TB, H), lambda i: (i, 0)),
            pl.BlockSpec((H, 3 * H), lambda i: (0, 0)),
            pl.BlockSpec((H, 3 * H), lambda i: (0, 0)),
            pl.BlockSpec((1, 3 * H), lambda i: (0, 0)),
            pl.BlockSpec((1, 3 * H), lambda i: (0, 0)),
        ],
        out_specs=pl.BlockSpec((TB, H), lambda i: (i, 0)),
        out_shape=jax.ShapeDtypeStruct((SC_BASE, H), jnp.float32),
    )(edge_embedding, nft, a_f, node_features[:SC_BASE],
      wih_t, whh_t, bih, bhh)

    return jnp.concatenate([out_tc, out_sc], axis=0)


# speedup vs baseline: 1.0326x; 1.0059x over previous
import functools
import jax
import jax.numpy as jnp
from jax import lax
from jax.experimental import pallas as pl
from jax.experimental.pallas import tpu as pltpu
from jax.experimental.pallas import tpu_sc as plsc

N = 1024
H = 64
B = 8
TB = 32
SC_BASE = 832
NW = 32
RPW = (N - SC_BASE) // NW
CH = 128
CPR = N // CH
TOTC = RPW * CPR

_mesh = plsc.VectorSubcoreMesh(core_axis_name="c", subcore_axis_name="s")


def _splat(x):
    return jnp.full((16,), x, jnp.int32)


@functools.partial(
    pl.kernel,
    out_type=jax.ShapeDtypeStruct(((N - SC_BASE) * H,), jnp.float32),
    mesh=_mesh,
    compiler_params=pltpu.CompilerParams(needs_layout_passes=False),
    scratch_types=[
        pltpu.VMEM((N * H,), jnp.float32),
        pltpu.VMEM((RPW * N,), jnp.int32),
        pltpu.VMEM((2, CH, H), jnp.float32),
        pltpu.VMEM((RPW * H,), jnp.float32),
        pltpu.SemaphoreType.DMA,
    ],
)
def _sc_agg(nf_hbm, adj_hbm, e_hbm, out_hbm,
            nf_v, adj_v, stage_v, out_v, sem):
    wid = lax.axis_index("s") * 2 + lax.axis_index("c")
    row0 = SC_BASE + wid * RPW

    dnums = lax.GatherDimensionNumbers(
        offset_dims=(), collapsed_slice_dims=(0,), start_index_map=(0,))

    def take16(v, idx):
        return lax.gather(v, idx[:, None], dnums, slice_sizes=(1,),
                          mode=lax.GatherScatterMode.PROMISE_IN_BOUNDS)

    pltpu.sync_copy(nf_hbm, nf_v)
    pltpu.sync_copy(adj_hbm.at[pl.ds((row0 - SC_BASE) * N, RPW * N)], adj_v)

    def _issue(gc):
        i_local = gc // CPR
        c = gc % CPR
        pltpu.async_copy(
            e_hbm.at[pl.ds((row0 + i_local) * N + c * CH, CH)],
            stage_v.at[gc % 2], sem)

    _issue(0)

    def chunk_body(gc, acc4):
        i_local = gc // CPR
        c = gc % CPR
        b = gc % 2
        lax.cond(gc + 1 < TOTC, lambda: _issue(gc + 1), lambda: None)
        pltpu.make_async_copy(
            e_hbm.at[pl.ds(0, CH)], stage_v.at[b], sem).wait()
        jbase = c * CH

        def eb_body(eb, acc4):
            wv = adj_v[pl.ds(i_local * N + jbase + eb * 16, 16)]
            wf = jnp.where(wv != 0, 1.0, 0.0).astype(jnp.float32)
            for k in range(16):
                w = take16(wf, _splat(k))
                e = eb * 16 + k
                j = jbase + e
                new = []
                for hb in range(4):
                    ev = stage_v[b, e, pl.ds(hb * 16, 16)]
                    nfv = nf_v[pl.ds(j * H + hb * 16, 16)]
                    new.append(acc4[hb] + w * ev * nfv)
                acc4 = tuple(new)
            return acc4

        acc4 = lax.fori_loop(0, CH // 16, eb_body, acc4)

        def flush(a4, i_local=i_local):
            for hb in range(4):
                out_v[pl.ds(i_local * H + hb * 16, 16)] = a4[hb]
            return (jnp.zeros((16,), jnp.float32),) * 4

        return lax.cond(c == CPR - 1, flush, lambda a4: a4, acc4)

    lax.fori_loop(0, TOTC, chunk_body,
                  (jnp.zeros((16,), jnp.float32),) * 4)

    pltpu.sync_copy(out_v, out_hbm.at[pl.ds((row0 - SC_BASE) * H, RPW * H)])


def _tc_kernel(e_ref, nft_ref, a_ref, h_ref,
               wih_ref, whh_ref, bih_ref, bhh_ref, out_ref):
    col = jax.lax.broadcasted_iota(jnp.int32, (B, B * N), 1) // N
    row = jax.lax.broadcasted_iota(jnp.int32, (B, B * N), 0)
    diag = (col == row).astype(jnp.float32)
    aggs = []
    for r0 in range(0, TB, B):
        q = e_ref[pl.ds(r0 * N, B * N), :] * nft_ref[:]
        aw = jnp.tile(a_ref[pl.ds(r0, B), :], (1, B))
        sa8 = jnp.where(aw != 0, diag, 0.0)
        aggs.append(jnp.dot(sa8, q, preferred_element_type=jnp.float32))
    agg = jnp.concatenate(aggs, axis=0)
    h = h_ref[:]
    gi = jnp.dot(agg, wih_ref[:], preferred_element_type=jnp.float32) + bih_ref[:]
    gh = jnp.dot(h, whh_ref[:], preferred_element_type=jnp.float32) + bhh_ref[:]
    r = jax.nn.sigmoid(gi[:, :H] + gh[:, :H])
    z = jax.nn.sigmoid(gi[:, H:2 * H] + gh[:, H:2 * H])
    n = jnp.tanh(gi[:, 2 * H:] + r * gh[:, 2 * H:])
    out_ref[:] = (1.0 - z) * n + z * h


def _gru_kernel(agg_ref, h_ref, wih_ref, whh_ref, bih_ref, bhh_ref, out_ref):
    gi = jnp.dot(agg_ref[:], wih_ref[:], preferred_element_type=jnp.float32) + bih_ref[:]
    gh = jnp.dot(h_ref[:], whh_ref[:], preferred_element_type=jnp.float32) + bhh_ref[:]
    r = jax.nn.sigmoid(gi[:, :H] + gh[:, :H])
    z = jax.nn.sigmoid(gi[:, H:2 * H] + gh[:, H:2 * H])
    n = jnp.tanh(gi[:, 2 * H:] + r * gh[:, 2 * H:])
    out_ref[:] = (1.0 - z) * n + z * h_ref[:]


def kernel(node_features, adjacency_matrix, edge_embedding,
           weight_ih, weight_hh, bias_ih, bias_hh):
    wih_t = weight_ih.T
    whh_t = weight_hh.T
    bih = bias_ih.reshape(1, 3 * H)
    bhh = bias_hh.reshape(1, 3 * H)

    agg_sc = _sc_agg(node_features.reshape(N * H),
                     adjacency_matrix[SC_BASE:].reshape((N - SC_BASE) * N),
                     edge_embedding).reshape(N - SC_BASE, H)
    out_sc = pl.pallas_call(
        _gru_kernel,
        out_shape=jax.ShapeDtypeStruct((N - SC_BASE, H), jnp.float32),
    )(agg_sc, node_features[SC_BASE:], wih_t, whh_t, bih, bhh)

    nft = jnp.tile(node_features, (B, 1))

    out_tc = pl.pallas_call(
        _tc_kernel,
        grid=(SC_BASE // TB,),
        in_specs=[
            pl.BlockSpec((TB * N, H), lambda i: (i, 0)),
            pl.BlockSpec((B * N, H), lambda i: (0, 0)),
            pl.BlockSpec((TB, N), lambda i: (i, 0)),
            pl.BlockSpec((TB, H), lambda i: (i, 0)),
            pl.BlockSpec((H, 3 * H), lambda i: (0, 0)),
            pl.BlockSpec((H, 3 * H), lambda i: (0, 0)),
            pl.BlockSpec((1, 3 * H), lambda i: (0, 0)),
            pl.BlockSpec((1, 3 * H), lambda i: (0, 0)),
        ],
        out_specs=pl.BlockSpec((TB, H), lambda i: (i, 0)),
        out_shape=jax.ShapeDtypeStruct((SC_BASE, H), jnp.float32),
    )(edge_embedding, nft, adjacency_matrix[:SC_BASE], node_features[:SC_BASE],
      wih_t, whh_t, bih, bhh)

    return jnp.concatenate([out_tc, out_sc], axis=0)


# speedup vs baseline: 1.0345x; 1.0018x over previous
import functools
import jax
import jax.numpy as jnp
from jax import lax
from jax.experimental import pallas as pl
from jax.experimental.pallas import tpu as pltpu
from jax.experimental.pallas import tpu_sc as plsc

N = 1024
H = 64
B = 8
TB = 32
SC_BASE = 864
NW = 32
RPW = (N - SC_BASE) // NW
CH = 128
CPR = N // CH
TOTC = RPW * CPR

_mesh = plsc.VectorSubcoreMesh(core_axis_name="c", subcore_axis_name="s")


def _splat(x):
    return jnp.full((16,), x, jnp.int32)


@functools.partial(
    pl.kernel,
    out_type=jax.ShapeDtypeStruct(((N - SC_BASE) * H,), jnp.float32),
    mesh=_mesh,
    compiler_params=pltpu.CompilerParams(needs_layout_passes=False),
    scratch_types=[
        pltpu.VMEM((N * H,), jnp.float32),
        pltpu.VMEM((RPW * N,), jnp.int32),
        pltpu.VMEM((2, CH, H), jnp.float32),
        pltpu.VMEM((RPW * H,), jnp.float32),
        pltpu.SemaphoreType.DMA,
    ],
)
def _sc_agg(nf_hbm, adj_hbm, e_hbm, out_hbm,
            nf_v, adj_v, stage_v, out_v, sem):
    wid = lax.axis_index("s") * 2 + lax.axis_index("c")
    row0 = SC_BASE + wid * RPW

    dnums = lax.GatherDimensionNumbers(
        offset_dims=(), collapsed_slice_dims=(0,), start_index_map=(0,))

    def take16(v, idx):
        return lax.gather(v, idx[:, None], dnums, slice_sizes=(1,),
                          mode=lax.GatherScatterMode.PROMISE_IN_BOUNDS)

    pltpu.sync_copy(nf_hbm, nf_v)
    pltpu.sync_copy(adj_hbm.at[pl.ds((row0 - SC_BASE) * N, RPW * N)], adj_v)

    def _issue(gc):
        i_local = gc // CPR
        c = gc % CPR
        pltpu.async_copy(
            e_hbm.at[pl.ds((row0 + i_local) * N + c * CH, CH)],
            stage_v.at[gc % 2], sem)

    _issue(0)

    def chunk_body(gc, acc4):
        i_local = gc // CPR
        c = gc % CPR
        b = gc % 2
        lax.cond(gc + 1 < TOTC, lambda: _issue(gc + 1), lambda: None)
        pltpu.make_async_copy(
            e_hbm.at[pl.ds(0, CH)], stage_v.at[b], sem).wait()
        jbase = c * CH

        def eb_body(eb, acc4):
            wv = adj_v[pl.ds(i_local * N + jbase + eb * 16, 16)]
            wf = jnp.where(wv != 0, 1.0, 0.0).astype(jnp.float32)
            for k in range(16):
                w = take16(wf, _splat(k))
                e = eb * 16 + k
                j = jbase + e
                new = []
                for hb in range(4):
                    ev = stage_v[b, e, pl.ds(hb * 16, 16)]
                    nfv = nf_v[pl.ds(j * H + hb * 16, 16)]
                    new.append(acc4[hb] + w * ev * nfv)
                acc4 = tuple(new)
            return acc4

        acc4 = lax.fori_loop(0, CH // 16, eb_body, acc4)

        def flush(a4, i_local=i_local):
            for hb in range(4):
                out_v[pl.ds(i_local * H + hb * 16, 16)] = a4[hb]
            return (jnp.zeros((16,), jnp.float32),) * 4

        return lax.cond(c == CPR - 1, flush, lambda a4: a4, acc4)

    lax.fori_loop(0, TOTC, chunk_body,
                  (jnp.zeros((16,), jnp.float32),) * 4)

    pltpu.sync_copy(out_v, out_hbm.at[pl.ds((row0 - SC_BASE) * H, RPW * H)])


def _tc_kernel(e_ref, nft_ref, a_ref, h_ref,
               wih_ref, whh_ref, bih_ref, bhh_ref, out_ref):
    col = jax.lax.broadcasted_iota(jnp.int32, (B, B * N), 1) // N
    row = jax.lax.broadcasted_iota(jnp.int32, (B, B * N), 0)
    diag = (col == row).astype(jnp.float32)
    aggs = []
    for r0 in range(0, TB, B):
        q = e_ref[pl.ds(r0 * N, B * N), :] * nft_ref[:]
        aw = jnp.tile(a_ref[pl.ds(r0, B), :], (1, B))
        sa8 = jnp.where(aw != 0, diag, 0.0)
        aggs.append(jnp.dot(sa8, q, preferred_element_type=jnp.float32))
    agg = jnp.concatenate(aggs, axis=0)
    h = h_ref[:]
    gi = jnp.dot(agg, wih_ref[:], preferred_element_type=jnp.float32) + bih_ref[:]
    gh = jnp.dot(h, whh_ref[:], preferred_element_type=jnp.float32) + bhh_ref[:]
    r = jax.nn.sigmoid(gi[:, :H] + gh[:, :H])
    z = jax.nn.sigmoid(gi[:, H:2 * H] + gh[:, H:2 * H])
    n = jnp.tanh(gi[:, 2 * H:] + r * gh[:, 2 * H:])
    out_ref[:] = (1.0 - z) * n + z * h


def _gru_kernel(agg_ref, h_ref, wih_ref, whh_ref, bih_ref, bhh_ref, out_ref):
    gi = jnp.dot(agg_ref[:], wih_ref[:], preferred_element_type=jnp.float32) + bih_ref[:]
    gh = jnp.dot(h_ref[:], whh_ref[:], preferred_element_type=jnp.float32) + bhh_ref[:]
    r = jax.nn.sigmoid(gi[:, :H] + gh[:, :H])
    z = jax.nn.sigmoid(gi[:, H:2 * H] + gh[:, H:2 * H])
    n = jnp.tanh(gi[:, 2 * H:] + r * gh[:, 2 * H:])
    out_ref[:] = (1.0 - z) * n + z * h_ref[:]


def kernel(node_features, adjacency_matrix, edge_embedding,
           weight_ih, weight_hh, bias_ih, bias_hh):
    wih_t = weight_ih.T
    whh_t = weight_hh.T
    bih = bias_ih.reshape(1, 3 * H)
    bhh = bias_hh.reshape(1, 3 * H)

    agg_sc = _sc_agg(node_features.reshape(N * H),
                     adjacency_matrix[SC_BASE:].reshape((N - SC_BASE) * N),
                     edge_embedding).reshape(N - SC_BASE, H)
    out_sc = pl.pallas_call(
        _gru_kernel,
        out_shape=jax.ShapeDtypeStruct((N - SC_BASE, H), jnp.float32),
    )(agg_sc, node_features[SC_BASE:], wih_t, whh_t, bih, bhh)

    nft = jnp.tile(node_features, (B, 1))

    out_tc = pl.pallas_call(
        _tc_kernel,
        grid=(SC_BASE // TB,),
        in_specs=[
            pl.BlockSpec((TB * N, H), lambda i: (i, 0)),
            pl.BlockSpec((B * N, H), lambda i: (0, 0)),
            pl.BlockSpec((TB, N), lambda i: (i, 0)),
            pl.BlockSpec((TB, H), lambda i: (i, 0)),
            pl.BlockSpec((H, 3 * H), lambda i: (0, 0)),
            pl.BlockSpec((H, 3 * H), lambda i: (0, 0)),
            pl.BlockSpec((1, 3 * H), lambda i: (0, 0)),
            pl.BlockSpec((1, 3 * H), lambda i: (0, 0)),
        ],
        out_specs=pl.BlockSpec((TB, H), lambda i: (i, 0)),
        out_shape=jax.ShapeDtypeStruct((SC_BASE, H), jnp.float32),
    )(edge_embedding, nft, adjacency_matrix[:SC_BASE], node_features[:SC_BASE],
      wih_t, whh_t, bih, bhh)

    return jnp.concatenate([out_tc, out_sc], axis=0)


# speedup vs baseline: 1.0367x; 1.0021x over previous
import functools
import jax
import jax.numpy as jnp
from jax import lax
from jax.experimental import pallas as pl
from jax.experimental.pallas import tpu as pltpu
from jax.experimental.pallas import tpu_sc as plsc

N = 1024
H = 64
B = 8
TB = 48
SC_BASE = 864
NW = 32
RPW = (N - SC_BASE) // NW
CH = 128
CPR = N // CH
TOTC = RPW * CPR

_mesh = plsc.VectorSubcoreMesh(core_axis_name="c", subcore_axis_name="s")


def _splat(x):
    return jnp.full((16,), x, jnp.int32)


@functools.partial(
    pl.kernel,
    out_type=jax.ShapeDtypeStruct(((N - SC_BASE) * H,), jnp.float32),
    mesh=_mesh,
    compiler_params=pltpu.CompilerParams(needs_layout_passes=False),
    scratch_types=[
        pltpu.VMEM((N * H,), jnp.float32),
        pltpu.VMEM((RPW * N,), jnp.int32),
        pltpu.VMEM((2, CH, H), jnp.float32),
        pltpu.VMEM((RPW * H,), jnp.float32),
        pltpu.SemaphoreType.DMA,
    ],
)
def _sc_agg(nf_hbm, adj_hbm, e_hbm, out_hbm,
            nf_v, adj_v, stage_v, out_v, sem):
    wid = lax.axis_index("s") * 2 + lax.axis_index("c")
    row0 = SC_BASE + wid * RPW

    dnums = lax.GatherDimensionNumbers(
        offset_dims=(), collapsed_slice_dims=(0,), start_index_map=(0,))

    def take16(v, idx):
        return lax.gather(v, idx[:, None], dnums, slice_sizes=(1,),
                          mode=lax.GatherScatterMode.PROMISE_IN_BOUNDS)

    pltpu.sync_copy(nf_hbm, nf_v)
    pltpu.sync_copy(adj_hbm.at[pl.ds((row0 - SC_BASE) * N, RPW * N)], adj_v)

    def _issue(gc):
        i_local = gc // CPR
        c = gc % CPR
        pltpu.async_copy(
            e_hbm.at[pl.ds((row0 + i_local) * N + c * CH, CH)],
            stage_v.at[gc % 2], sem)

    _issue(0)

    def chunk_body(gc, acc4):
        i_local = gc // CPR
        c = gc % CPR
        b = gc % 2
        lax.cond(gc + 1 < TOTC, lambda: _issue(gc + 1), lambda: None)
        pltpu.make_async_copy(
            e_hbm.at[pl.ds(0, CH)], stage_v.at[b], sem).wait()
        jbase = c * CH

        def eb_body(eb, acc4):
            wv = adj_v[pl.ds(i_local * N + jbase + eb * 16, 16)]
            wf = jnp.where(wv != 0, 1.0, 0.0).astype(jnp.float32)
            for k in range(16):
                w = take16(wf, _splat(k))
                e = eb * 16 + k
                j = jbase + e
                new = []
                for hb in range(4):
                    ev = stage_v[b, e, pl.ds(hb * 16, 16)]
                    nfv = nf_v[pl.ds(j * H + hb * 16, 16)]
                    new.append(acc4[hb] + w * ev * nfv)
                acc4 = tuple(new)
            return acc4

        acc4 = lax.fori_loop(0, CH // 16, eb_body, acc4)

        def flush(a4, i_local=i_local):
            for hb in range(4):
                out_v[pl.ds(i_local * H + hb * 16, 16)] = a4[hb]
            return (jnp.zeros((16,), jnp.float32),) * 4

        return lax.cond(c == CPR - 1, flush, lambda a4: a4, acc4)

    lax.fori_loop(0, TOTC, chunk_body,
                  (jnp.zeros((16,), jnp.float32),) * 4)

    pltpu.sync_copy(out_v, out_hbm.at[pl.ds((row0 - SC_BASE) * H, RPW * H)])


def _tc_kernel(e_ref, nft_ref, a_ref, h_ref,
               wih_ref, whh_ref, bih_ref, bhh_ref, out_ref):
    col = jax.lax.broadcasted_iota(jnp.int32, (B, B * N), 1) // N
    row = jax.lax.broadcasted_iota(jnp.int32, (B, B * N), 0)
    diag = (col == row).astype(jnp.float32)
    aggs = []
    for r0 in range(0, TB, B):
        q = e_ref[pl.ds(r0 * N, B * N), :] * nft_ref[:]
        aw = jnp.tile(a_ref[pl.ds(r0, B), :], (1, B))
        sa8 = jnp.where(aw != 0, diag, 0.0)
        aggs.append(jnp.dot(sa8, q, preferred_element_type=jnp.float32))
    agg = jnp.concatenate(aggs, axis=0)
    h = h_ref[:]
    gi = jnp.dot(agg, wih_ref[:], preferred_element_type=jnp.float32) + bih_ref[:]
    gh = jnp.dot(h, whh_ref[:], preferred_element_type=jnp.float32) + bhh_ref[:]
    r = jax.nn.sigmoid(gi[:, :H] + gh[:, :H])
    z = jax.nn.sigmoid(gi[:, H:2 * H] + gh[:, H:2 * H])
    n = jnp.tanh(gi[:, 2 * H:] + r * gh[:, 2 * H:])
    out_ref[:] = (1.0 - z) * n + z * h


def _gru_kernel(agg_ref, h_ref, wih_ref, whh_ref, bih_ref, bhh_ref, out_ref):
    gi = jnp.dot(agg_ref[:], wih_ref[:], preferred_element_type=jnp.float32) + bih_ref[:]
    gh = jnp.dot(h_ref[:], whh_ref[:], preferred_element_type=jnp.float32) + bhh_ref[:]
    r = jax.nn.sigmoid(gi[:, :H] + gh[:, :H])
    z = jax.nn.sigmoid(gi[:, H:2 * H] + gh[:, H:2 * H])
    n = jnp.tanh(gi[:, 2 * H:] + r * gh[:, 2 * H:])
    out_ref[:] = (1.0 - z) * n + z * h_ref[:]


def kernel(node_features, adjacency_matrix, edge_embedding,
           weight_ih, weight_hh, bias_ih, bias_hh):
    wih_t = weight_ih.T
    whh_t = weight_hh.T
    bih = bias_ih.reshape(1, 3 * H)
    bhh = bias_hh.reshape(1, 3 * H)

    agg_sc = _sc_agg(node_features.reshape(N * H),
                     adjacency_matrix[SC_BASE:].reshape((N - SC_BASE) * N),
                     edge_embedding).reshape(N - SC_BASE, H)
    out_sc = pl.pallas_call(
        _gru_kernel,
        out_shape=jax.ShapeDtypeStruct((N - SC_BASE, H), jnp.float32),
    )(agg_sc, node_features[SC_BASE:], wih_t, whh_t, bih, bhh)

    nft = jnp.tile(node_features, (B, 1))

    out_tc = pl.pallas_call(
        _tc_kernel,
        grid=(SC_BASE // TB,),
        in_specs=[
            pl.BlockSpec((TB * N, H), lambda i: (i, 0)),
            pl.BlockSpec((B * N, H), lambda i: (0, 0)),
            pl.BlockSpec((TB, N), lambda i: (i, 0)),
            pl.BlockSpec((TB, H), lambda i: (i, 0)),
            pl.BlockSpec((H, 3 * H), lambda i: (0, 0)),
            pl.BlockSpec((H, 3 * H), lambda i: (0, 0)),
            pl.BlockSpec((1, 3 * H), lambda i: (0, 0)),
            pl.BlockSpec((1, 3 * H), lambda i: (0, 0)),
        ],
        out_specs=pl.BlockSpec((TB, H), lambda i: (i, 0)),
        out_shape=jax.ShapeDtypeStruct((SC_BASE, H), jnp.float32),
    )(edge_embedding, nft, adjacency_matrix[:SC_BASE], node_features[:SC_BASE],
      wih_t, whh_t, bih, bhh)

    return jnp.concatenate([out_tc, out_sc], axis=0)
